# baseline (speedup 1.0000x reference)
import functools

import jax
import jax.numpy as jnp
from jax import lax
from jax.experimental import pallas as pl
from jax.experimental.pallas import tpu as pltpu
from jax.experimental.pallas import tpu_sc as plsc

N = 10000
E = 320000
H = 128
DE = 16
V = 26
OUT = 128

NP = 10240
ROWS_PER_TILE = NP // 16
NW = 32
CH = 128
ITERS = 80
EWP = CH * ITERS
EP = EWP * NW
RB = 2
NB = ITERS // RB

_EPS = 1e-5


def _embed_body(x_ref, emb_ref, w_ref, b_ref, g_ref, bb_ref, o_ref):
    bn = x_ref.shape[0]
    idx = lax.broadcasted_iota(jnp.int32, (bn, 128), 1)
    onehot = (idx == x_ref[...]).astype(jnp.float32)
    he = jnp.dot(onehot, emb_ref[...], preferred_element_type=jnp.float32)
    h = jnp.dot(jnp.maximum(he, 0.0), w_ref[...],
                preferred_element_type=jnp.float32) + b_ref[...]
    mu = jnp.mean(h, axis=-1, keepdims=True)
    var = jnp.mean((h - mu) ** 2, axis=-1, keepdims=True)
    o_ref[...] = (h - mu) * lax.rsqrt(var + _EPS) * g_ref[...] + bb_ref[...]


def _node_embed(x2, emb_pad, w, b, g, bb):
    BN = 1000
    return pl.pallas_call(
        _embed_body,
        grid=(N // BN,),
        in_specs=[
            pl.BlockSpec((BN, 1), lambda i: (i, 0)),
            pl.BlockSpec((128, H), lambda i: (0, 0)),
            pl.BlockSpec((H, H), lambda i: (0, 0)),
            pl.BlockSpec((1, H), lambda i: (0, 0)),
            pl.BlockSpec((1, H), lambda i: (0, 0)),
            pl.BlockSpec((1, H), lambda i: (0, 0)),
        ],
        out_specs=pl.BlockSpec((BN, H), lambda i: (i, 0)),
        out_shape=jax.ShapeDtypeStruct((N, H), jnp.float32),
    )(x2, emb_pad, w, b, g, bb)


def _edge_embed_body(a_ref, w1_ref, b1_ref, w2_ref, b2_ref, g_ref, bb_ref, o_ref):
    h1 = jnp.maximum(
        jnp.dot(a_ref[...], w1_ref[...], preferred_element_type=jnp.float32)
        + b1_ref[...], 0.0)
    h = jnp.dot(h1, w2_ref[...], preferred_element_type=jnp.float32) + b2_ref[...]
    mu = jnp.mean(h, axis=-1, keepdims=True)
    var = jnp.mean((h - mu) ** 2, axis=-1, keepdims=True)
    o_ref[...] = (h - mu) * lax.rsqrt(var + _EPS) * g_ref[...] + bb_ref[...]


def _edge_embed(edge_attr, w1, b1, w2, b2, g, bb):
    BE = 4000
    return pl.pallas_call(
        _edge_embed_body,
        grid=(E // BE,),
        in_specs=[
            pl.BlockSpec((BE, DE), lambda i: (i, 0)),
            pl.BlockSpec((DE, H), lambda i: (0, 0)),
            pl.BlockSpec((1, H), lambda i: (0, 0)),
            pl.BlockSpec((H, H), lambda i: (0, 0)),
            pl.BlockSpec((1, H), lambda i: (0, 0)),
            pl.BlockSpec((1, H), lambda i: (0, 0)),
            pl.BlockSpec((1, H), lambda i: (0, 0)),
        ],
        out_specs=pl.BlockSpec((BE, H), lambda i: (i, 0)),
        out_shape=jax.ShapeDtypeStruct((E, H), jnp.float32),
    )(edge_attr, w1, b1, w2, b2, g, bb)


def _mlp(xd, xs, ea, w1d_ref, w1s_ref, w1e_ref, b1_ref, w2_ref, b2_ref):
    h1 = (jnp.dot(xd, w1d_ref[...], preferred_element_type=jnp.float32)
          + jnp.dot(xs, w1s_ref[...], preferred_element_type=jnp.float32)
          + jnp.dot(ea, w1e_ref[...], preferred_element_type=jnp.float32)
          + b1_ref[...])
    h1 = jnp.maximum(h1, 0.0)
    return jnp.dot(h1, w2_ref[...], preferred_element_type=jnp.float32) + b2_ref[...]


def _acc_stats(m, s1_ref, s2_ref):
    @pl.when(pl.program_id(0) == 0)
    def _():
        s1_ref[...] = jnp.zeros_like(s1_ref)
        s2_ref[...] = jnp.zeros_like(s2_ref)

    s1_ref[...] += jnp.sum(m, axis=0, keepdims=True)
    s2_ref[...] += jnp.sum(m * m, axis=0, keepdims=True)


def _bn_apply(mp, ps1_ref, ps2_ref, g_ref, bb_ref):
    mu = ps1_ref[...] * (1.0 / E)
    var = ps2_ref[...] * (1.0 / E) - mu * mu
    rstd = lax.rsqrt(var + _EPS)
    return (mp - mu) * rstd * g_ref[...] + bb_ref[...]


def _edge_conv1_body(xd_ref, xs_ref, ea_ref, w1d_ref, w1s_ref, w1e_ref, b1_ref,
                     w2_ref, b2_ref, m_ref, s1_ref, s2_ref):
    m = _mlp(xd_ref[...], xs_ref[...], ea_ref[...],
             w1d_ref, w1s_ref, w1e_ref, b1_ref, w2_ref, b2_ref)
    m_ref[...] = m
    _acc_stats(m, s1_ref, s2_ref)


def _edge_conv_mid_body(xd_ref, xs_ref, ea_ref, mp_ref, ps1_ref, ps2_ref,
                        g_ref, bb_ref, w1d_ref, w1s_ref, w1e_ref, b1_ref,
                        w2_ref, b2_ref, eao_ref, m_ref, s1_ref, s2_ref):
    ean = jnp.maximum(
        ea_ref[...] + _bn_apply(mp_ref[...], ps1_ref, ps2_ref, g_ref, bb_ref),
        0.0)
    eao_ref[...] = ean
    m = _mlp(xd_ref[...], xs_ref[...], ean,
             w1d_ref, w1s_ref, w1e_ref, b1_ref, w2_ref, b2_ref)
    m_ref[...] = m
    _acc_stats(m, s1_ref, s2_ref)


def _edge_conv_last_body(xd_ref, xs_ref, ea_ref, mp_ref, ps1_ref, ps2_ref,
                         g_ref, bb_ref, w1d_ref, w1s_ref, w1e_ref, b1_ref,
                         w2_ref, b2_ref, m_ref):
    ean = jnp.maximum(
        ea_ref[...] + _bn_apply(mp_ref[...], ps1_ref, ps2_ref, g_ref, bb_ref),
        0.0)
    m_ref[...] = _mlp(xd_ref[...], xs_ref[...], ean,
                      w1d_ref, w1s_ref, w1e_ref, b1_ref, w2_ref, b2_ref)


_BE = 4000
_EDGE = pl.BlockSpec((_BE, H), lambda i: (i, 0))
_ROW = pl.BlockSpec((1, H), lambda i: (0, 0))
_ROW2 = pl.BlockSpec((1, 2 * H), lambda i: (0, 0))
_W1B = pl.BlockSpec((H, 2 * H), lambda i: (0, 0))
_W2B = pl.BlockSpec((2 * H, H), lambda i: (0, 0))
_M_OUT = jax.ShapeDtypeStruct((EP, H), jnp.float32)
_S_OUT = jax.ShapeDtypeStruct((1, H), jnp.float32)


def _edge_conv1(xd, xs, ea, w1d, w1s, w1e, b1, w2, b2):
    return pl.pallas_call(
        _edge_conv1_body,
        grid=(E // _BE,),
        in_specs=[_EDGE, _EDGE, _EDGE, _W1B, _W1B, _W1B, _ROW2, _W2B, _ROW],
        out_specs=[_EDGE, _ROW, _ROW],
        out_shape=[_M_OUT, _S_OUT, _S_OUT],
    )(xd, xs, ea, w1d, w1s, w1e, b1, w2, b2)


def _edge_conv_mid(xd, xs, ea, mp, ps1, ps2, g, bb, w1d, w1s, w1e, b1, w2, b2):
    return pl.pallas_call(
        _edge_conv_mid_body,
        grid=(E // _BE,),
        in_specs=[_EDGE, _EDGE, _EDGE, _EDGE, _ROW, _ROW, _ROW, _ROW,
                  _W1B, _W1B, _W1B, _ROW2, _W2B, _ROW],
        out_specs=[_EDGE, _EDGE, _ROW, _ROW],
        out_shape=[_M_OUT, _M_OUT, _S_OUT, _S_OUT],
    )(xd, xs, ea, mp, ps1, ps2, g, bb, w1d, w1s, w1e, b1, w2, b2)


def _edge_conv_last(xd, xs, ea, mp, ps1, ps2, g, bb, w1d, w1s, w1e, b1, w2, b2):
    return pl.pallas_call(
        _edge_conv_last_body,
        grid=(E // _BE,),
        in_specs=[_EDGE, _EDGE, _EDGE, _EDGE, _ROW, _ROW, _ROW, _ROW,
                  _W1B, _W1B, _W1B, _ROW2, _W2B, _ROW],
        out_specs=_EDGE,
        out_shape=_M_OUT,
    )(xd, xs, ea, mp, ps1, ps2, g, bb, w1d, w1s, w1e, b1, w2, b2)


def _node_update_body(p_ref, h_ref, g_ref, bb_ref, o_ref, *, last, wo_ref=None,
                      bo_ref=None):
    xo = p_ref[0, :N, :] + p_ref[1, :N, :]
    mu = jnp.mean(xo, axis=0, keepdims=True)
    var = jnp.mean((xo - mu) ** 2, axis=0, keepdims=True)
    xo = (xo - mu) * lax.rsqrt(var + _EPS) * g_ref[...] + bb_ref[...]
    h = h_ref[...] + xo
    if last:
        o_ref[...] = jnp.dot(h, wo_ref[...],
                             preferred_element_type=jnp.float32) + bo_ref[...]
    else:
        o_ref[...] = jnp.maximum(h, 0.0)


def _mid_node_body(p_ref, h_ref, g_ref, bb_ref, o_ref):
    _node_update_body(p_ref, h_ref, g_ref, bb_ref, o_ref, last=False)


def _last_node_body(p_ref, h_ref, g_ref, bb_ref, wo_ref, bo_ref, o_ref):
    _node_update_body(p_ref, h_ref, g_ref, bb_ref, o_ref, last=True,
                      wo_ref=wo_ref, bo_ref=bo_ref)


def _node_update(p, h, g, bb, wo=None, bo=None):
    last = wo is not None
    args = [p, h, g, bb]
    if last:
        args += [wo, bo]
    return pl.pallas_call(
        _last_node_body if last else _mid_node_body,
        out_shape=jax.ShapeDtypeStruct((N, OUT if last else H), jnp.float32),
    )(*args)


def _sc_gather_body(h_hbm, dst_hbm, src_hbm, xd_hbm, xs_hbm,
                    idx_d, idx_s, d1, s1, db, sb, gsem, wsem):
    c = lax.axis_index("c")
    s = lax.axis_index("s")
    wid = s * 2 + c
    dbuf = (db,)
    sbuf = (sb,)

    pltpu.sync_copy(dst_hbm.at[wid], idx_d)
    pltpu.sync_copy(src_hbm.at[wid], idx_s)

    def body(j, carry):
        base = wid * EWP + j * CH
        for k in range(CH // 16):
            d1[pl.ds(16 * k, 16)] = idx_d[j, pl.ds(16 * k, 16)]
            s1[pl.ds(16 * k, 16)] = idx_s[j, pl.ds(16 * k, 16)]
        pltpu.async_copy(h_hbm.at[d1], dbuf[0], gsem)
        pltpu.async_copy(h_hbm.at[s1], sbuf[0], gsem)
        pltpu.make_async_copy(h_hbm.at[d1], dbuf[0], gsem).wait()
        pltpu.make_async_copy(h_hbm.at[s1], sbuf[0], gsem).wait()
        pltpu.async_copy(dbuf[0], xd_hbm.at[pl.ds(base, CH)], wsem)
        pltpu.async_copy(sbuf[0], xs_hbm.at[pl.ds(base, CH)], wsem)
        pltpu.make_async_copy(dbuf[0], xd_hbm.at[pl.ds(base, CH)], wsem).wait()
        pltpu.make_async_copy(sbuf[0], xs_hbm.at[pl.ds(base, CH)], wsem).wait()
        return carry

    lax.fori_loop(0, ITERS, body, 0)


def _sc_scatter_body(m_hbm, dstp_hbm, out_hbm, idx_v, m0, m1, zbuf,
                     acc, lsem, asem):
    c = lax.axis_index("c")
    s = lax.axis_index("s")
    wid = s * 2 + c
    mbuf = (m0, m1)
    SRB = len(mbuf)

    zero = jnp.zeros((16,), jnp.float32)
    for r in range(16):
        for k in range(H // 16):
            zbuf[r, pl.ds(16 * k, 16)] = zero
    for j in range(ROWS_PER_TILE // 16):
        pltpu.sync_copy(zbuf, acc.at[pl.ds(s * ROWS_PER_TILE + j * 16, 16)])

    pltpu.sync_copy(dstp_hbm.at[wid], idx_v)
    plsc.subcore_barrier()

    def body(jj, carry):
        j0 = jj * SRB

        @pl.when(jj > 0)
        def _():
            for b in range(SRB):
                pltpu.make_async_copy(
                    mbuf[b], acc.at[pl.ds(0, CH)], asem).wait()

        for b in range(SRB):
            base = wid * EWP + (j0 + b) * CH
            pltpu.async_copy(m_hbm.at[pl.ds(base, CH)], mbuf[b], lsem)
        for b in range(SRB):
            pltpu.make_async_copy(
                m_hbm.at[pl.ds(0, CH)], mbuf[b], lsem).wait()
            pltpu.async_copy(mbuf[b], acc.at[idx_v.at[j0 + b]], asem, add=True)
        return carry

    lax.fori_loop(0, ITERS // SRB, body, 0)
    for b in range(SRB):
        pltpu.make_async_copy(mbuf[b], acc.at[pl.ds(0, CH)], asem).wait()
    plsc.subcore_barrier()
    pltpu.sync_copy(acc.at[pl.ds(s * ROWS_PER_TILE, ROWS_PER_TILE)],
                    out_hbm.at[c, pl.ds(s * ROWS_PER_TILE, ROWS_PER_TILE)])


@functools.cache
def _sc_kernels():
    mesh = plsc.VectorSubcoreMesh(core_axis_name="c", subcore_axis_name="s",
                                  num_cores=2, num_subcores=16)
    gather = pl.kernel(
        _sc_gather_body,
        out_type=[jax.ShapeDtypeStruct((EP, H), jnp.float32),
                  jax.ShapeDtypeStruct((EP, H), jnp.float32)],
        mesh=mesh,
        scratch_types=[
            pltpu.VMEM((ITERS, CH), jnp.int32),
            pltpu.VMEM((ITERS, CH), jnp.int32),
            pltpu.VMEM((CH,), jnp.int32),
            pltpu.VMEM((CH,), jnp.int32),
            pltpu.VMEM((CH, H), jnp.float32),
            pltpu.VMEM((CH, H), jnp.float32),
            pltpu.SemaphoreType.DMA,
            pltpu.SemaphoreType.DMA,
        ],
    )
    scatter = pl.kernel(
        _sc_scatter_body,
        out_type=jax.ShapeDtypeStruct((2, NP, H), jnp.float32),
        mesh=mesh,
        scratch_types=[
            pltpu.VMEM((ITERS, CH), jnp.int32),
            pltpu.VMEM((CH, H), jnp.float32),
            pltpu.VMEM((CH, H), jnp.float32),
            pltpu.VMEM((16, H), jnp.float32),
            pltpu.VMEM_SHARED((NP, H), jnp.float32),
            pltpu.SemaphoreType.DMA,
            pltpu.SemaphoreType.DMA,
        ],
    )
    return gather, scatter


def _sc_gather(h, dst3, src3):
    return _sc_kernels()[0](h, dst3, src3)


def _sc_scatter(m, dstp3):
    return _sc_kernels()[1](m, dstp3)


def kernel(x, edge_index, edge_attr, params):
    src = edge_index[0]
    dst = edge_index[1]
    dstp = jnp.where(src == dst, jnp.int32(N), dst).astype(jnp.int32)
    pad = ((0, EP - E),)
    dst3 = jnp.pad(dst, pad).reshape(NW, ITERS, CH)
    src3 = jnp.pad(src, pad).reshape(NW, ITERS, CH)
    dstp3 = jnp.pad(dstp, pad, constant_values=N).reshape(NW, ITERS, CH)
    x2 = x.reshape(N, 1).astype(jnp.int32)

    pe = params["embed_x"]
    emb_pad = jnp.zeros((128, H), jnp.float32).at[:V].set(pe["emb"])
    h = _node_embed(x2, emb_pad, pe["W"], pe["b"].reshape(1, H),
                    pe["ln_g"].reshape(1, H), pe["ln_b"].reshape(1, H))

    pa = params["embed_adj"]
    ea = _edge_embed(edge_attr, pa["W1"], pa["b1"].reshape(1, H),
                     pa["W2"], pa["b2"].reshape(1, H),
                     pa["ln_g"].reshape(1, H), pa["ln_b"].reshape(1, H))

    m = s1 = s2 = None
    prev = None
    for li, name in enumerate(["gc1", "gc2", "gc3", "gc4"]):
        p = params[name]
        w1 = p["W1"]
        wargs = (w1[:H], w1[H:2 * H], w1[2 * H:],
                 p["b1"].reshape(1, 2 * H), p["W2"], p["b2"].reshape(1, H))
        xd, xs = _sc_gather(h, dst3, src3)
        if li == 0:
            m, s1, s2 = _edge_conv1(xd, xs, ea, *wargs)
        elif li < 3:
            ea, m, s1, s2 = _edge_conv_mid(
                xd, xs, ea, m, s1, s2,
                prev["bne_g"].reshape(1, H), prev["bne_b"].reshape(1, H),
                *wargs)
        else:
            m = _edge_conv_last(
                xd, xs, ea, m, s1, s2,
                prev["bne_g"].reshape(1, H), prev["bne_b"].reshape(1, H),
                *wargs)
        part = _sc_scatter(m, dstp3)
        if li < 3:
            h = _node_update(part, h, p["bnx_g"].reshape(1, H),
                             p["bnx_b"].reshape(1, H))
        else:
            out = _node_update(part, h, p["bnx_g"].reshape(1, H),
                               p["bnx_b"].reshape(1, H),
                               params["out"]["W"],
                               params["out"]["b"].reshape(1, OUT))
        prev = p
    return out


# speedup vs baseline: 2.0630x; 2.0630x over previous
import functools

import jax
import jax.numpy as jnp
from jax import lax
from jax.experimental import pallas as pl
from jax.experimental.pallas import tpu as pltpu
from jax.experimental.pallas import tpu_sc as plsc

N = 10000
E = 320000
H = 128
DE = 16
V = 26
OUT = 128

NP = 10240
ROWS_PER_TILE = NP // 16
NW = 32
CH = 128
ITERS = 80
EWP = CH * ITERS
EP = EWP * NW
RB = 2
NB = ITERS // RB

_EPS = 1e-5


def _embed_body(x_ref, emb_ref, w_ref, b_ref, g_ref, bb_ref, o_ref):
    bn = x_ref.shape[0]
    idx = lax.broadcasted_iota(jnp.int32, (bn, 128), 1)
    onehot = (idx == x_ref[...]).astype(jnp.float32)
    he = jnp.dot(onehot, emb_ref[...], preferred_element_type=jnp.float32)
    h = jnp.dot(jnp.maximum(he, 0.0), w_ref[...],
                preferred_element_type=jnp.float32) + b_ref[...]
    mu = jnp.mean(h, axis=-1, keepdims=True)
    var = jnp.mean((h - mu) ** 2, axis=-1, keepdims=True)
    o_ref[...] = (h - mu) * lax.rsqrt(var + _EPS) * g_ref[...] + bb_ref[...]


def _node_embed(x2, emb_pad, w, b, g, bb):
    BN = 1000
    return pl.pallas_call(
        _embed_body,
        grid=(N // BN,),
        in_specs=[
            pl.BlockSpec((BN, 1), lambda i: (i, 0)),
            pl.BlockSpec((128, H), lambda i: (0, 0)),
            pl.BlockSpec((H, H), lambda i: (0, 0)),
            pl.BlockSpec((1, H), lambda i: (0, 0)),
            pl.BlockSpec((1, H), lambda i: (0, 0)),
            pl.BlockSpec((1, H), lambda i: (0, 0)),
        ],
        out_specs=pl.BlockSpec((BN, H), lambda i: (i, 0)),
        out_shape=jax.ShapeDtypeStruct((N, H), jnp.float32),
    )(x2, emb_pad, w, b, g, bb)


def _edge_embed_body(a_ref, w1_ref, b1_ref, w2_ref, b2_ref, g_ref, bb_ref, o_ref):
    h1 = jnp.maximum(
        jnp.dot(a_ref[...], w1_ref[...], preferred_element_type=jnp.float32)
        + b1_ref[...], 0.0)
    h = jnp.dot(h1, w2_ref[...], preferred_element_type=jnp.float32) + b2_ref[...]
    mu = jnp.mean(h, axis=-1, keepdims=True)
    var = jnp.mean((h - mu) ** 2, axis=-1, keepdims=True)
    o_ref[...] = (h - mu) * lax.rsqrt(var + _EPS) * g_ref[...] + bb_ref[...]


def _edge_embed(edge_attr, w1, b1, w2, b2, g, bb):
    BE = 4000
    return pl.pallas_call(
        _edge_embed_body,
        grid=(E // BE,),
        in_specs=[
            pl.BlockSpec((BE, DE), lambda i: (i, 0)),
            pl.BlockSpec((DE, H), lambda i: (0, 0)),
            pl.BlockSpec((1, H), lambda i: (0, 0)),
            pl.BlockSpec((H, H), lambda i: (0, 0)),
            pl.BlockSpec((1, H), lambda i: (0, 0)),
            pl.BlockSpec((1, H), lambda i: (0, 0)),
            pl.BlockSpec((1, H), lambda i: (0, 0)),
        ],
        out_specs=pl.BlockSpec((BE, H), lambda i: (i, 0)),
        out_shape=jax.ShapeDtypeStruct((E, H), jnp.float32),
    )(edge_attr, w1, b1, w2, b2, g, bb)


def _mlp(xd, xs, ea, w1d_ref, w1s_ref, w1e_ref, b1_ref, w2_ref, b2_ref):
    h1 = (jnp.dot(xd, w1d_ref[...], preferred_element_type=jnp.float32)
          + jnp.dot(xs, w1s_ref[...], preferred_element_type=jnp.float32)
          + jnp.dot(ea, w1e_ref[...], preferred_element_type=jnp.float32)
          + b1_ref[...])
    h1 = jnp.maximum(h1, 0.0)
    return jnp.dot(h1, w2_ref[...], preferred_element_type=jnp.float32) + b2_ref[...]


def _acc_stats(m, s1_ref, s2_ref):
    @pl.when(pl.program_id(0) == 0)
    def _():
        s1_ref[...] = jnp.zeros_like(s1_ref)
        s2_ref[...] = jnp.zeros_like(s2_ref)

    s1_ref[...] += jnp.sum(m, axis=0, keepdims=True)
    s2_ref[...] += jnp.sum(m * m, axis=0, keepdims=True)


def _bn_apply(mp, ps1_ref, ps2_ref, g_ref, bb_ref):
    mu = ps1_ref[...] * (1.0 / E)
    var = ps2_ref[...] * (1.0 / E) - mu * mu
    rstd = lax.rsqrt(var + _EPS)
    return (mp - mu) * rstd * g_ref[...] + bb_ref[...]


def _edge_conv1_body(xd_ref, xs_ref, ea_ref, w1d_ref, w1s_ref, w1e_ref, b1_ref,
                     w2_ref, b2_ref, m_ref, s1_ref, s2_ref):
    m = _mlp(xd_ref[...], xs_ref[...], ea_ref[...],
             w1d_ref, w1s_ref, w1e_ref, b1_ref, w2_ref, b2_ref)
    m_ref[...] = m
    _acc_stats(m, s1_ref, s2_ref)


def _edge_conv_mid_body(xd_ref, xs_ref, ea_ref, mp_ref, ps1_ref, ps2_ref,
                        g_ref, bb_ref, w1d_ref, w1s_ref, w1e_ref, b1_ref,
                        w2_ref, b2_ref, eao_ref, m_ref, s1_ref, s2_ref):
    ean = jnp.maximum(
        ea_ref[...] + _bn_apply(mp_ref[...], ps1_ref, ps2_ref, g_ref, bb_ref),
        0.0)
    eao_ref[...] = ean
    m = _mlp(xd_ref[...], xs_ref[...], ean,
             w1d_ref, w1s_ref, w1e_ref, b1_ref, w2_ref, b2_ref)
    m_ref[...] = m
    _acc_stats(m, s1_ref, s2_ref)


def _edge_conv_last_body(xd_ref, xs_ref, ea_ref, mp_ref, ps1_ref, ps2_ref,
                         g_ref, bb_ref, w1d_ref, w1s_ref, w1e_ref, b1_ref,
                         w2_ref, b2_ref, m_ref):
    ean = jnp.maximum(
        ea_ref[...] + _bn_apply(mp_ref[...], ps1_ref, ps2_ref, g_ref, bb_ref),
        0.0)
    m_ref[...] = _mlp(xd_ref[...], xs_ref[...], ean,
                      w1d_ref, w1s_ref, w1e_ref, b1_ref, w2_ref, b2_ref)


_BE = 4000
_EDGE = pl.BlockSpec((_BE, H), lambda i: (i, 0))
_ROW = pl.BlockSpec((1, H), lambda i: (0, 0))
_ROW2 = pl.BlockSpec((1, 2 * H), lambda i: (0, 0))
_W1B = pl.BlockSpec((H, 2 * H), lambda i: (0, 0))
_W2B = pl.BlockSpec((2 * H, H), lambda i: (0, 0))
_M_OUT = jax.ShapeDtypeStruct((EP, H), jnp.float32)
_S_OUT = jax.ShapeDtypeStruct((1, H), jnp.float32)


def _edge_conv1(xd, xs, ea, w1d, w1s, w1e, b1, w2, b2):
    return pl.pallas_call(
        _edge_conv1_body,
        grid=(E // _BE,),
        in_specs=[_EDGE, _EDGE, _EDGE, _W1B, _W1B, _W1B, _ROW2, _W2B, _ROW],
        out_specs=[_EDGE, _ROW, _ROW],
        out_shape=[_M_OUT, _S_OUT, _S_OUT],
    )(xd, xs, ea, w1d, w1s, w1e, b1, w2, b2)


def _edge_conv_mid(xd, xs, ea, mp, ps1, ps2, g, bb, w1d, w1s, w1e, b1, w2, b2):
    return pl.pallas_call(
        _edge_conv_mid_body,
        grid=(E // _BE,),
        in_specs=[_EDGE, _EDGE, _EDGE, _EDGE, _ROW, _ROW, _ROW, _ROW,
                  _W1B, _W1B, _W1B, _ROW2, _W2B, _ROW],
        out_specs=[_EDGE, _EDGE, _ROW, _ROW],
        out_shape=[_M_OUT, _M_OUT, _S_OUT, _S_OUT],
    )(xd, xs, ea, mp, ps1, ps2, g, bb, w1d, w1s, w1e, b1, w2, b2)


def _edge_conv_last(xd, xs, ea, mp, ps1, ps2, g, bb, w1d, w1s, w1e, b1, w2, b2):
    return pl.pallas_call(
        _edge_conv_last_body,
        grid=(E // _BE,),
        in_specs=[_EDGE, _EDGE, _EDGE, _EDGE, _ROW, _ROW, _ROW, _ROW,
                  _W1B, _W1B, _W1B, _ROW2, _W2B, _ROW],
        out_specs=_EDGE,
        out_shape=_M_OUT,
    )(xd, xs, ea, mp, ps1, ps2, g, bb, w1d, w1s, w1e, b1, w2, b2)


def _node_update_body(p_ref, h_ref, g_ref, bb_ref, o_ref, *, last, wo_ref=None,
                      bo_ref=None):
    xo = p_ref[0, :N, :] + p_ref[1, :N, :]
    mu = jnp.mean(xo, axis=0, keepdims=True)
    var = jnp.mean((xo - mu) ** 2, axis=0, keepdims=True)
    xo = (xo - mu) * lax.rsqrt(var + _EPS) * g_ref[...] + bb_ref[...]
    h = h_ref[...] + xo
    if last:
        o_ref[...] = jnp.dot(h, wo_ref[...],
                             preferred_element_type=jnp.float32) + bo_ref[...]
    else:
        o_ref[...] = jnp.maximum(h, 0.0)


def _mid_node_body(p_ref, h_ref, g_ref, bb_ref, o_ref):
    _node_update_body(p_ref, h_ref, g_ref, bb_ref, o_ref, last=False)


def _last_node_body(p_ref, h_ref, g_ref, bb_ref, wo_ref, bo_ref, o_ref):
    _node_update_body(p_ref, h_ref, g_ref, bb_ref, o_ref, last=True,
                      wo_ref=wo_ref, bo_ref=bo_ref)


def _node_update(p, h, g, bb, wo=None, bo=None):
    last = wo is not None
    args = [p, h, g, bb]
    if last:
        args += [wo, bo]
    return pl.pallas_call(
        _last_node_body if last else _mid_node_body,
        out_shape=jax.ShapeDtypeStruct((N, OUT if last else H), jnp.float32),
    )(*args)


def _sc_gather_body(h_hbm, dst_hbm, src_hbm, xd_hbm, xs_hbm,
                    idx_d, idx_s, d1, s1, db, sb, gsem, wsem):
    c = lax.axis_index("c")
    s = lax.axis_index("s")
    wid = s * 2 + c
    dbuf = (db,)
    sbuf = (sb,)

    pltpu.sync_copy(dst_hbm.at[wid], idx_d)
    pltpu.sync_copy(src_hbm.at[wid], idx_s)

    def body(j, carry):
        base = wid * EWP + j * CH
        for k in range(CH // 16):
            d1[pl.ds(16 * k, 16)] = idx_d[j, pl.ds(16 * k, 16)]
            s1[pl.ds(16 * k, 16)] = idx_s[j, pl.ds(16 * k, 16)]
        pltpu.async_copy(h_hbm.at[d1], dbuf[0], gsem)
        pltpu.async_copy(h_hbm.at[s1], sbuf[0], gsem)
        pltpu.make_async_copy(h_hbm.at[d1], dbuf[0], gsem).wait()
        pltpu.make_async_copy(h_hbm.at[s1], sbuf[0], gsem).wait()
        pltpu.async_copy(dbuf[0], xd_hbm.at[pl.ds(base, CH)], wsem)
        pltpu.async_copy(sbuf[0], xs_hbm.at[pl.ds(base, CH)], wsem)
        pltpu.make_async_copy(dbuf[0], xd_hbm.at[pl.ds(base, CH)], wsem).wait()
        pltpu.make_async_copy(sbuf[0], xs_hbm.at[pl.ds(base, CH)], wsem).wait()
        return carry

    lax.fori_loop(0, ITERS, body, 0)


def _sc_scatter_body(m_hbm, dstp_hbm, out_hbm, idx_v, m0, m1, zbuf,
                     acc, lsem, asem):
    c = lax.axis_index("c")
    s = lax.axis_index("s")
    wid = s * 2 + c
    mbuf = (m0, m1)
    SRB = len(mbuf)

    zero = jnp.zeros((16,), jnp.float32)
    for r in range(16):
        for k in range(H // 16):
            zbuf[r, pl.ds(16 * k, 16)] = zero
    for j in range(ROWS_PER_TILE // 16):
        pltpu.sync_copy(zbuf, acc.at[pl.ds(s * ROWS_PER_TILE + j * 16, 16)])

    pltpu.sync_copy(dstp_hbm.at[wid], idx_v)
    plsc.subcore_barrier()

    def body(jj, carry):
        j0 = jj * SRB

        @pl.when(jj > 0)
        def _():
            for b in range(SRB):
                pltpu.make_async_copy(
                    mbuf[b], acc.at[pl.ds(0, CH)], asem).wait()

        for b in range(SRB):
            base = wid * EWP + (j0 + b) * CH
            pltpu.async_copy(m_hbm.at[pl.ds(base, CH)], mbuf[b], lsem)
        for b in range(SRB):
            pltpu.make_async_copy(
                m_hbm.at[pl.ds(0, CH)], mbuf[b], lsem).wait()
            pltpu.async_copy(mbuf[b], acc.at[idx_v.at[j0 + b]], asem, add=True)
        return carry

    lax.fori_loop(0, ITERS // SRB, body, 0)
    for b in range(SRB):
        pltpu.make_async_copy(mbuf[b], acc.at[pl.ds(0, CH)], asem).wait()
    plsc.subcore_barrier()
    pltpu.sync_copy(acc.at[pl.ds(s * ROWS_PER_TILE, ROWS_PER_TILE)],
                    out_hbm.at[c, pl.ds(s * ROWS_PER_TILE, ROWS_PER_TILE)])


@functools.cache
def _sc_kernels():
    mesh = plsc.VectorSubcoreMesh(core_axis_name="c", subcore_axis_name="s",
                                  num_cores=2, num_subcores=16)
    gather = pl.kernel(
        _sc_gather_body,
        out_type=[jax.ShapeDtypeStruct((EP, H), jnp.float32),
                  jax.ShapeDtypeStruct((EP, H), jnp.float32)],
        mesh=mesh,
        scratch_types=[
            pltpu.VMEM((ITERS, CH), jnp.int32),
            pltpu.VMEM((ITERS, CH), jnp.int32),
            pltpu.VMEM((CH,), jnp.int32),
            pltpu.VMEM((CH,), jnp.int32),
            pltpu.VMEM((CH, H), jnp.float32),
            pltpu.VMEM((CH, H), jnp.float32),
            pltpu.SemaphoreType.DMA,
            pltpu.SemaphoreType.DMA,
        ],
    )
    scatter = pl.kernel(
        _sc_scatter_body,
        out_type=jax.ShapeDtypeStruct((2, NP, H), jnp.float32),
        mesh=mesh,
        scratch_types=[
            pltpu.VMEM((ITERS, CH), jnp.int32),
            pltpu.VMEM((CH, H), jnp.float32),
            pltpu.VMEM((CH, H), jnp.float32),
            pltpu.VMEM((16, H), jnp.float32),
            pltpu.VMEM_SHARED((NP, H), jnp.float32),
            pltpu.SemaphoreType.DMA,
            pltpu.SemaphoreType.DMA,
        ],
    )
    return gather, scatter


def _sc_gather(h, dst3, src3):
    return _sc_kernels()[0](h, dst3, src3)


def _sc_scatter(m, dstp3):
    return _sc_kernels()[1](m, dstp3)


def kernel(x, edge_index, edge_attr, params):
    src = edge_index[0]
    dst = edge_index[1]
    dstp = jnp.where(src == dst, jnp.int32(N), dst).astype(jnp.int32)
    pad = ((0, EP - E),)
    spread = (jnp.arange(EP - E, dtype=jnp.int32) * 13) % N
    dst3 = jnp.concatenate([dst, spread]).reshape(NW, ITERS, CH)
    src3 = jnp.concatenate([src, spread]).reshape(NW, ITERS, CH)
    dstp3 = jnp.pad(dstp, pad, constant_values=N).reshape(NW, ITERS, CH)
    x2 = x.reshape(N, 1).astype(jnp.int32)

    pe = params["embed_x"]
    emb_pad = jnp.zeros((128, H), jnp.float32).at[:V].set(pe["emb"])
    h = _node_embed(x2, emb_pad, pe["W"], pe["b"].reshape(1, H),
                    pe["ln_g"].reshape(1, H), pe["ln_b"].reshape(1, H))

    pa = params["embed_adj"]
    ea = _edge_embed(edge_attr, pa["W1"], pa["b1"].reshape(1, H),
                     pa["W2"], pa["b2"].reshape(1, H),
                     pa["ln_g"].reshape(1, H), pa["ln_b"].reshape(1, H))

    m = s1 = s2 = None
    prev = None
    for li, name in enumerate(["gc1", "gc2", "gc3", "gc4"]):
        p = params[name]
        w1 = p["W1"]
        wargs = (w1[:H], w1[H:2 * H], w1[2 * H:],
                 p["b1"].reshape(1, 2 * H), p["W2"], p["b2"].reshape(1, H))
        xd, xs = _sc_gather(h, dst3, src3)
        if li == 0:
            m, s1, s2 = _edge_conv1(xd, xs, ea, *wargs)
        elif li < 3:
            ea, m, s1, s2 = _edge_conv_mid(
                xd, xs, ea, m, s1, s2,
                prev["bne_g"].reshape(1, H), prev["bne_b"].reshape(1, H),
                *wargs)
        else:
            m = _edge_conv_last(
                xd, xs, ea, m, s1, s2,
                prev["bne_g"].reshape(1, H), prev["bne_b"].reshape(1, H),
                *wargs)
        part = _sc_scatter(m, dstp3)
        if li < 3:
            h = _node_update(part, h, p["bnx_g"].reshape(1, H),
                             p["bnx_b"].reshape(1, H))
        else:
            out = _node_update(part, h, p["bnx_g"].reshape(1, H),
                               p["bnx_b"].reshape(1, H),
                               params["out"]["W"],
                               params["out"]["b"].reshape(1, OUT))
        prev = p
    return out


# speedup vs baseline: 2.1700x; 1.0519x over previous
import functools

import jax
import jax.numpy as jnp
from jax import lax
from jax.experimental import pallas as pl
from jax.experimental.pallas import tpu as pltpu
from jax.experimental.pallas import tpu_sc as plsc

N = 10000
E = 320000
H = 128
DE = 16
V = 26
OUT = 128

NP = 10240
ROWS_PER_TILE = NP // 16
NW = 32
CH = 128
ITERS = 80
EWP = CH * ITERS
EP = EWP * NW
RB = 2
NB = ITERS // RB

_EPS = 1e-5


def _embed_body(x_ref, emb_ref, w_ref, b_ref, g_ref, bb_ref, o_ref):
    bn = x_ref.shape[0]
    idx = lax.broadcasted_iota(jnp.int32, (bn, 128), 1)
    onehot = (idx == x_ref[...]).astype(jnp.float32)
    he = jnp.dot(onehot, emb_ref[...], preferred_element_type=jnp.float32)
    h = jnp.dot(jnp.maximum(he, 0.0), w_ref[...],
                preferred_element_type=jnp.float32) + b_ref[...]
    mu = jnp.mean(h, axis=-1, keepdims=True)
    var = jnp.mean((h - mu) ** 2, axis=-1, keepdims=True)
    o_ref[...] = (h - mu) * lax.rsqrt(var + _EPS) * g_ref[...] + bb_ref[...]


def _node_embed(x2, emb_pad, w, b, g, bb):
    BN = 1000
    return pl.pallas_call(
        _embed_body,
        grid=(N // BN,),
        in_specs=[
            pl.BlockSpec((BN, 1), lambda i: (i, 0)),
            pl.BlockSpec((128, H), lambda i: (0, 0)),
            pl.BlockSpec((H, H), lambda i: (0, 0)),
            pl.BlockSpec((1, H), lambda i: (0, 0)),
            pl.BlockSpec((1, H), lambda i: (0, 0)),
            pl.BlockSpec((1, H), lambda i: (0, 0)),
        ],
        out_specs=pl.BlockSpec((BN, H), lambda i: (i, 0)),
        out_shape=jax.ShapeDtypeStruct((N, H), jnp.float32),
    )(x2, emb_pad, w, b, g, bb)


def _edge_embed_body(a_ref, w1_ref, b1_ref, w2_ref, b2_ref, g_ref, bb_ref, o_ref):
    h1 = jnp.maximum(
        jnp.dot(a_ref[...], w1_ref[...], preferred_element_type=jnp.float32)
        + b1_ref[...], 0.0)
    h = jnp.dot(h1, w2_ref[...], preferred_element_type=jnp.float32) + b2_ref[...]
    mu = jnp.mean(h, axis=-1, keepdims=True)
    var = jnp.mean((h - mu) ** 2, axis=-1, keepdims=True)
    o_ref[...] = (h - mu) * lax.rsqrt(var + _EPS) * g_ref[...] + bb_ref[...]


def _edge_embed(edge_attr, w1, b1, w2, b2, g, bb):
    BE = 4000
    return pl.pallas_call(
        _edge_embed_body,
        grid=(E // BE,),
        in_specs=[
            pl.BlockSpec((BE, DE), lambda i: (i, 0)),
            pl.BlockSpec((DE, H), lambda i: (0, 0)),
            pl.BlockSpec((1, H), lambda i: (0, 0)),
            pl.BlockSpec((H, H), lambda i: (0, 0)),
            pl.BlockSpec((1, H), lambda i: (0, 0)),
            pl.BlockSpec((1, H), lambda i: (0, 0)),
            pl.BlockSpec((1, H), lambda i: (0, 0)),
        ],
        out_specs=pl.BlockSpec((BE, H), lambda i: (i, 0)),
        out_shape=jax.ShapeDtypeStruct((E, H), jnp.float32),
    )(edge_attr, w1, b1, w2, b2, g, bb)


def _mlp(xd, xs, ea, w1d_ref, w1s_ref, w1e_ref, b1_ref, w2_ref, b2_ref):
    h1 = (jnp.dot(xd, w1d_ref[...], preferred_element_type=jnp.float32)
          + jnp.dot(xs, w1s_ref[...], preferred_element_type=jnp.float32)
          + jnp.dot(ea, w1e_ref[...], preferred_element_type=jnp.float32)
          + b1_ref[...])
    h1 = jnp.maximum(h1, 0.0)
    return jnp.dot(h1, w2_ref[...], preferred_element_type=jnp.float32) + b2_ref[...]


def _acc_stats(m, s1_ref, s2_ref):
    @pl.when(pl.program_id(0) == 0)
    def _():
        s1_ref[...] = jnp.zeros_like(s1_ref)
        s2_ref[...] = jnp.zeros_like(s2_ref)

    s1_ref[...] += jnp.sum(m, axis=0, keepdims=True)
    s2_ref[...] += jnp.sum(m * m, axis=0, keepdims=True)


def _bn_apply(mp, ps1_ref, ps2_ref, g_ref, bb_ref):
    mu = ps1_ref[...] * (1.0 / E)
    var = ps2_ref[...] * (1.0 / E) - mu * mu
    rstd = lax.rsqrt(var + _EPS)
    return (mp - mu) * rstd * g_ref[...] + bb_ref[...]


def _edge_conv1_body(xd_ref, xs_ref, ea_ref, w1d_ref, w1s_ref, w1e_ref, b1_ref,
                     w2_ref, b2_ref, m_ref, s1_ref, s2_ref):
    m = _mlp(xd_ref[...], xs_ref[...], ea_ref[...],
             w1d_ref, w1s_ref, w1e_ref, b1_ref, w2_ref, b2_ref)
    m_ref[...] = m
    _acc_stats(m, s1_ref, s2_ref)


def _edge_conv_mid_body(xd_ref, xs_ref, ea_ref, mp_ref, ps1_ref, ps2_ref,
                        g_ref, bb_ref, w1d_ref, w1s_ref, w1e_ref, b1_ref,
                        w2_ref, b2_ref, eao_ref, m_ref, s1_ref, s2_ref):
    ean = jnp.maximum(
        ea_ref[...] + _bn_apply(mp_ref[...], ps1_ref, ps2_ref, g_ref, bb_ref),
        0.0)
    eao_ref[...] = ean
    m = _mlp(xd_ref[...], xs_ref[...], ean,
             w1d_ref, w1s_ref, w1e_ref, b1_ref, w2_ref, b2_ref)
    m_ref[...] = m
    _acc_stats(m, s1_ref, s2_ref)


def _edge_conv_last_body(xd_ref, xs_ref, ea_ref, mp_ref, ps1_ref, ps2_ref,
                         g_ref, bb_ref, w1d_ref, w1s_ref, w1e_ref, b1_ref,
                         w2_ref, b2_ref, m_ref):
    ean = jnp.maximum(
        ea_ref[...] + _bn_apply(mp_ref[...], ps1_ref, ps2_ref, g_ref, bb_ref),
        0.0)
    m_ref[...] = _mlp(xd_ref[...], xs_ref[...], ean,
                      w1d_ref, w1s_ref, w1e_ref, b1_ref, w2_ref, b2_ref)


_BE = 4000
_EDGE = pl.BlockSpec((_BE, H), lambda i: (i, 0))
_ROW = pl.BlockSpec((1, H), lambda i: (0, 0))
_ROW2 = pl.BlockSpec((1, 2 * H), lambda i: (0, 0))
_W1B = pl.BlockSpec((H, 2 * H), lambda i: (0, 0))
_W2B = pl.BlockSpec((2 * H, H), lambda i: (0, 0))
_M_OUT = jax.ShapeDtypeStruct((EP, H), jnp.float32)
_S_OUT = jax.ShapeDtypeStruct((1, H), jnp.float32)


def _edge_conv1(xd, xs, ea, w1d, w1s, w1e, b1, w2, b2):
    return pl.pallas_call(
        _edge_conv1_body,
        grid=(E // _BE,),
        in_specs=[_EDGE, _EDGE, _EDGE, _W1B, _W1B, _W1B, _ROW2, _W2B, _ROW],
        out_specs=[_EDGE, _ROW, _ROW],
        out_shape=[_M_OUT, _S_OUT, _S_OUT],
    )(xd, xs, ea, w1d, w1s, w1e, b1, w2, b2)


def _edge_conv_mid(xd, xs, ea, mp, ps1, ps2, g, bb, w1d, w1s, w1e, b1, w2, b2):
    return pl.pallas_call(
        _edge_conv_mid_body,
        grid=(E // _BE,),
        in_specs=[_EDGE, _EDGE, _EDGE, _EDGE, _ROW, _ROW, _ROW, _ROW,
                  _W1B, _W1B, _W1B, _ROW2, _W2B, _ROW],
        out_specs=[_EDGE, _EDGE, _ROW, _ROW],
        out_shape=[_M_OUT, _M_OUT, _S_OUT, _S_OUT],
    )(xd, xs, ea, mp, ps1, ps2, g, bb, w1d, w1s, w1e, b1, w2, b2)


def _edge_conv_last(xd, xs, ea, mp, ps1, ps2, g, bb, w1d, w1s, w1e, b1, w2, b2):
    return pl.pallas_call(
        _edge_conv_last_body,
        grid=(E // _BE,),
        in_specs=[_EDGE, _EDGE, _EDGE, _EDGE, _ROW, _ROW, _ROW, _ROW,
                  _W1B, _W1B, _W1B, _ROW2, _W2B, _ROW],
        out_specs=_EDGE,
        out_shape=_M_OUT,
    )(xd, xs, ea, mp, ps1, ps2, g, bb, w1d, w1s, w1e, b1, w2, b2)


def _node_update_body(p_ref, h_ref, g_ref, bb_ref, o_ref, *, last, wo_ref=None,
                      bo_ref=None):
    xo = p_ref[0, :N, :] + p_ref[1, :N, :]
    mu = jnp.mean(xo, axis=0, keepdims=True)
    var = jnp.mean((xo - mu) ** 2, axis=0, keepdims=True)
    xo = (xo - mu) * lax.rsqrt(var + _EPS) * g_ref[...] + bb_ref[...]
    h = h_ref[...] + xo
    if last:
        o_ref[...] = jnp.dot(h, wo_ref[...],
                             preferred_element_type=jnp.float32) + bo_ref[...]
    else:
        o_ref[...] = jnp.maximum(h, 0.0)


def _mid_node_body(p_ref, h_ref, g_ref, bb_ref, o_ref):
    _node_update_body(p_ref, h_ref, g_ref, bb_ref, o_ref, last=False)


def _last_node_body(p_ref, h_ref, g_ref, bb_ref, wo_ref, bo_ref, o_ref):
    _node_update_body(p_ref, h_ref, g_ref, bb_ref, o_ref, last=True,
                      wo_ref=wo_ref, bo_ref=bo_ref)


def _node_update(p, h, g, bb, wo=None, bo=None):
    last = wo is not None
    args = [p, h, g, bb]
    if last:
        args += [wo, bo]
    return pl.pallas_call(
        _last_node_body if last else _mid_node_body,
        out_shape=jax.ShapeDtypeStruct((N, OUT if last else H), jnp.float32),
    )(*args)


def _sc_gather_body(h_hbm, dst_hbm, src_hbm, xd_hbm, xs_hbm,
                    idx_d, idx_s, d0, d1, s0, s1, gsem, wsem):
    c = lax.axis_index("c")
    s = lax.axis_index("s")
    wid = s * 2 + c
    dbuf = (d0, d1)
    sbuf = (s0, s1)

    pltpu.sync_copy(dst_hbm.at[wid], idx_d)
    pltpu.sync_copy(src_hbm.at[wid], idx_s)

    def body(jj, carry):
        j0 = jj * RB

        @pl.when(jj > 0)
        def _():
            for b in range(RB):
                pltpu.make_async_copy(dbuf[b], xd_hbm.at[pl.ds(0, CH)], wsem).wait()
                pltpu.make_async_copy(sbuf[b], xs_hbm.at[pl.ds(0, CH)], wsem).wait()

        for b in range(RB):
            j = j0 + b
            pltpu.async_copy(h_hbm.at[idx_d.at[j]], dbuf[b], gsem)
            pltpu.async_copy(h_hbm.at[idx_s.at[j]], sbuf[b], gsem)
        for b in range(RB):
            j = j0 + b
            pltpu.make_async_copy(h_hbm.at[idx_d.at[j]], dbuf[b], gsem).wait()
            pltpu.make_async_copy(h_hbm.at[idx_s.at[j]], sbuf[b], gsem).wait()
            base = wid * EWP + j * CH
            pltpu.async_copy(dbuf[b], xd_hbm.at[pl.ds(base, CH)], wsem)
            pltpu.async_copy(sbuf[b], xs_hbm.at[pl.ds(base, CH)], wsem)
        return carry

    lax.fori_loop(0, NB, body, 0)
    for b in range(RB):
        pltpu.make_async_copy(dbuf[b], xd_hbm.at[pl.ds(0, CH)], wsem).wait()
        pltpu.make_async_copy(sbuf[b], xs_hbm.at[pl.ds(0, CH)], wsem).wait()


def _sc_scatter_body(m_hbm, dstp_hbm, out_hbm, idx_v, m0, m1, zbuf,
                     acc, lsem, asem):
    c = lax.axis_index("c")
    s = lax.axis_index("s")
    wid = s * 2 + c
    mbuf = (m0, m1)
    SRB = len(mbuf)

    zero = jnp.zeros((16,), jnp.float32)
    for r in range(16):
        for k in range(H // 16):
            zbuf[r, pl.ds(16 * k, 16)] = zero
    for j in range(ROWS_PER_TILE // 16):
        pltpu.sync_copy(zbuf, acc.at[pl.ds(s * ROWS_PER_TILE + j * 16, 16)])

    pltpu.sync_copy(dstp_hbm.at[wid], idx_v)
    plsc.subcore_barrier()

    def body(jj, carry):
        j0 = jj * SRB

        @pl.when(jj > 0)
        def _():
            for b in range(SRB):
                pltpu.make_async_copy(
                    mbuf[b], acc.at[pl.ds(0, CH)], asem).wait()

        for b in range(SRB):
            base = wid * EWP + (j0 + b) * CH
            pltpu.async_copy(m_hbm.at[pl.ds(base, CH)], mbuf[b], lsem)
        for b in range(SRB):
            pltpu.make_async_copy(
                m_hbm.at[pl.ds(0, CH)], mbuf[b], lsem).wait()
            pltpu.async_copy(mbuf[b], acc.at[idx_v.at[j0 + b]], asem, add=True)
        return carry

    lax.fori_loop(0, ITERS // SRB, body, 0)
    for b in range(SRB):
        pltpu.make_async_copy(mbuf[b], acc.at[pl.ds(0, CH)], asem).wait()
    plsc.subcore_barrier()
    pltpu.sync_copy(acc.at[pl.ds(s * ROWS_PER_TILE, ROWS_PER_TILE)],
                    out_hbm.at[c, pl.ds(s * ROWS_PER_TILE, ROWS_PER_TILE)])


@functools.cache
def _sc_kernels():
    mesh = plsc.VectorSubcoreMesh(core_axis_name="c", subcore_axis_name="s",
                                  num_cores=2, num_subcores=16)
    gather = pl.kernel(
        _sc_gather_body,
        out_type=[jax.ShapeDtypeStruct((EP, H), jnp.float32),
                  jax.ShapeDtypeStruct((EP, H), jnp.float32)],
        mesh=mesh,
        scratch_types=[
            pltpu.VMEM((ITERS, CH), jnp.int32),
            pltpu.VMEM((ITERS, CH), jnp.int32),
            pltpu.VMEM((CH, H), jnp.float32),
            pltpu.VMEM((CH, H), jnp.float32),
            pltpu.VMEM((CH, H), jnp.float32),
            pltpu.VMEM((CH, H), jnp.float32),
            pltpu.SemaphoreType.DMA,
            pltpu.SemaphoreType.DMA,
        ],
    )
    scatter = pl.kernel(
        _sc_scatter_body,
        out_type=jax.ShapeDtypeStruct((2, NP, H), jnp.float32),
        mesh=mesh,
        scratch_types=[
            pltpu.VMEM((ITERS, CH), jnp.int32),
            pltpu.VMEM((CH, H), jnp.float32),
            pltpu.VMEM((CH, H), jnp.float32),
            pltpu.VMEM((16, H), jnp.float32),
            pltpu.VMEM_SHARED((NP, H), jnp.float32),
            pltpu.SemaphoreType.DMA,
            pltpu.SemaphoreType.DMA,
        ],
    )
    return gather, scatter


def _sc_gather(h, dst3, src3):
    return _sc_kernels()[0](h, dst3, src3)


def _sc_scatter(m, dstp3):
    return _sc_kernels()[1](m, dstp3)


def kernel(x, edge_index, edge_attr, params):
    src = edge_index[0]
    dst = edge_index[1]
    dstp = jnp.where(src == dst, jnp.int32(N), dst).astype(jnp.int32)
    pad = ((0, EP - E),)
    spread = (jnp.arange(EP - E, dtype=jnp.int32) * 13) % N
    dst3 = jnp.concatenate([dst, spread]).reshape(NW, ITERS, CH)
    src3 = jnp.concatenate([src, spread]).reshape(NW, ITERS, CH)
    dstp3 = jnp.pad(dstp, pad, constant_values=N).reshape(NW, ITERS, CH)
    x2 = x.reshape(N, 1).astype(jnp.int32)

    pe = params["embed_x"]
    emb_pad = jnp.zeros((128, H), jnp.float32).at[:V].set(pe["emb"])
    h = _node_embed(x2, emb_pad, pe["W"], pe["b"].reshape(1, H),
                    pe["ln_g"].reshape(1, H), pe["ln_b"].reshape(1, H))

    pa = params["embed_adj"]
    ea = _edge_embed(edge_attr, pa["W1"], pa["b1"].reshape(1, H),
                     pa["W2"], pa["b2"].reshape(1, H),
                     pa["ln_g"].reshape(1, H), pa["ln_b"].reshape(1, H))

    m = s1 = s2 = None
    prev = None
    for li, name in enumerate(["gc1", "gc2", "gc3", "gc4"]):
        p = params[name]
        w1 = p["W1"]
        wargs = (w1[:H], w1[H:2 * H], w1[2 * H:],
                 p["b1"].reshape(1, 2 * H), p["W2"], p["b2"].reshape(1, H))
        xd, xs = _sc_gather(h, dst3, src3)
        if li == 0:
            m, s1, s2 = _edge_conv1(xd, xs, ea, *wargs)
        elif li < 3:
            ea, m, s1, s2 = _edge_conv_mid(
                xd, xs, ea, m, s1, s2,
                prev["bne_g"].reshape(1, H), prev["bne_b"].reshape(1, H),
                *wargs)
        else:
            m = _edge_conv_last(
                xd, xs, ea, m, s1, s2,
                prev["bne_g"].reshape(1, H), prev["bne_b"].reshape(1, H),
                *wargs)
        part = _sc_scatter(m, dstp3)
        if li < 3:
            h = _node_update(part, h, p["bnx_g"].reshape(1, H),
                             p["bnx_b"].reshape(1, H))
        else:
            out = _node_update(part, h, p["bnx_g"].reshape(1, H),
                               p["bnx_b"].reshape(1, H),
                               params["out"]["W"],
                               params["out"]["b"].reshape(1, OUT))
        prev = p
    return out
